# baseline (speedup 1.0000x reference)
import functools
import math

import jax
import jax.numpy as jnp
from jax.experimental import pallas as pl
import jax.experimental.pallas.tpu as pltpu

_NUM_UNITS = 1024
_SCALE = math.sqrt(float(_NUM_UNITS))
_NEG2LN1E4 = -2.0 * math.log(10000.0) / float(_NUM_UNITS)


def _pe_tile_kernel(o_ref, vbuf, s_ref, c_ref, sem, *, tile_t, n_steps, n_batch):
    pid = pl.program_id(0)
    slot = jax.lax.rem(pid, 2)
    col = jax.lax.broadcasted_iota(jnp.int32, (1, _NUM_UNITS), 1)
    w = jnp.exp(col.astype(jnp.float32) * _NEG2LN1E4)

    @pl.when(pid == 0)
    def _build_lo_tables():
        sub = 16
        num_m = tile_t // sub
        r16 = jax.lax.broadcasted_iota(jnp.int32, (sub, _NUM_UNITS), 0)
        b = r16.astype(jnp.float32) * w
        sr = jnp.sin(b)
        cr = jnp.cos(b)
        mm = jax.lax.broadcasted_iota(jnp.int32, (num_m, _NUM_UNITS), 0)
        a = mm.astype(jnp.float32) * (w * float(sub))
        sm = jnp.sin(a)
        cm = jnp.cos(a)
        for m in range(num_m):
            smm = sm[m : m + 1, :]
            cmm = cm[m : m + 1, :]
            s_ref[m * sub : (m + 1) * sub, :] = smm * cr + cmm * sr
            c_ref[m * sub : (m + 1) * sub, :] = cmm * cr - smm * sr

    @pl.when(pid >= 2)
    def _wait_prev():
        for b in range(n_batch):
            pltpu.make_async_copy(
                vbuf.at[slot],
                o_ref.at[b, pl.ds((pid - 2) * tile_t, tile_t), :],
                sem.at[slot, b],
            ).wait()

    a_hi = (pid * tile_t).astype(jnp.float32) * w
    sh = jnp.sin(a_hi)
    ch = jnp.cos(a_hi)
    even = (col & 1) == 0
    p = jnp.where(even, sh, ch) * _SCALE
    q = jnp.where(even, ch, -sh) * _SCALE
    val = p * c_ref[...] + q * s_ref[...]
    vbuf[slot] = val

    @pl.when(pid == 0)
    def _zero_row0():
        vbuf[0, 0:1, :] = jnp.zeros((1, _NUM_UNITS), jnp.float32)

    for b in range(n_batch):
        pltpu.make_async_copy(
            vbuf.at[slot],
            o_ref.at[b, pl.ds(pid * tile_t, tile_t), :],
            sem.at[slot, b],
        ).start()

    @pl.when(pid == n_steps - 1)
    def _drain():
        for b in range(n_batch):
            pltpu.make_async_copy(
                vbuf.at[1 - slot],
                o_ref.at[b, pl.ds((pid - 1) * tile_t, tile_t), :],
                sem.at[1 - slot, b],
            ).wait()
            pltpu.make_async_copy(
                vbuf.at[slot],
                o_ref.at[b, pl.ds(pid * tile_t, tile_t), :],
                sem.at[slot, b],
            ).wait()


def kernel(inputs):
    n, t = inputs.shape
    tile_t = 512
    n_steps = t // tile_t
    out = pl.pallas_call(
        functools.partial(
            _pe_tile_kernel, tile_t=tile_t, n_steps=n_steps, n_batch=n
        ),
        grid=(n_steps,),
        out_specs=pl.BlockSpec(memory_space=pltpu.MemorySpace.HBM),
        out_shape=jax.ShapeDtypeStruct((n, t, _NUM_UNITS), jnp.float32),
        scratch_shapes=[
            pltpu.VMEM((2, tile_t, _NUM_UNITS), jnp.float32),
            pltpu.VMEM((tile_t, _NUM_UNITS), jnp.float32),
            pltpu.VMEM((tile_t, _NUM_UNITS), jnp.float32),
            pltpu.SemaphoreType.DMA((2, 4)),
        ],
    )()
    return out


# speedup vs baseline: 1.0555x; 1.0555x over previous
import functools
import math

import jax
import jax.numpy as jnp
from jax.experimental import pallas as pl
import jax.experimental.pallas.tpu as pltpu

_NUM_UNITS = 1024
_SCALE = math.sqrt(float(_NUM_UNITS))
_NEG2LN1E4 = -2.0 * math.log(10000.0) / float(_NUM_UNITS)


def _pe_tile_kernel(o_ref, vbuf, s_ref, c_ref, sem, *, tile_t, n_steps, n_batch):
    pid = pl.program_id(0)
    slot = jax.lax.rem(pid, 2)
    col = jax.lax.broadcasted_iota(jnp.int32, (1, _NUM_UNITS), 1)
    w = jnp.exp(col.astype(jnp.float32) * _NEG2LN1E4)

    @pl.when(pid == 0)
    def _build_lo_tables():
        sub = 16
        num_m = tile_t // sub
        r16 = jax.lax.broadcasted_iota(jnp.int32, (sub, _NUM_UNITS), 0)
        b = r16.astype(jnp.float32) * w
        sr = jnp.sin(b)
        cr = jnp.cos(b)
        mm = jax.lax.broadcasted_iota(jnp.int32, (num_m, _NUM_UNITS), 0)
        a = mm.astype(jnp.float32) * (w * float(sub))
        sm = jnp.sin(a)
        cm = jnp.cos(a)
        for m in range(num_m):
            smm = sm[m : m + 1, :]
            cmm = cm[m : m + 1, :]
            s_ref[m * sub : (m + 1) * sub, :] = smm * cr + cmm * sr
            c_ref[m * sub : (m + 1) * sub, :] = cmm * cr - smm * sr

    @pl.when(pid >= 2)
    def _wait_prev():
        for b in range(n_batch):
            pltpu.make_async_copy(
                vbuf.at[slot],
                o_ref.at[b, pl.ds((pid - 2) * tile_t, tile_t), :],
                sem.at[slot, b],
            ).wait()

    a_hi = (pid * tile_t).astype(jnp.float32) * w
    sh = jnp.sin(a_hi)
    ch = jnp.cos(a_hi)
    even = (col & 1) == 0
    p = jnp.where(even, sh, ch) * _SCALE
    q = jnp.where(even, ch, -sh) * _SCALE
    val = p * c_ref[...] + q * s_ref[...]
    vbuf[slot] = val

    @pl.when(pid == 0)
    def _zero_row0():
        vbuf[0, 0:1, :] = jnp.zeros((1, _NUM_UNITS), jnp.float32)

    for b in range(n_batch):
        pltpu.make_async_copy(
            vbuf.at[slot],
            o_ref.at[b, pl.ds(pid * tile_t, tile_t), :],
            sem.at[slot, b],
        ).start()

    @pl.when(pid == n_steps - 1)
    def _drain():
        for b in range(n_batch):
            pltpu.make_async_copy(
                vbuf.at[1 - slot],
                o_ref.at[b, pl.ds((pid - 1) * tile_t, tile_t), :],
                sem.at[1 - slot, b],
            ).wait()
            pltpu.make_async_copy(
                vbuf.at[slot],
                o_ref.at[b, pl.ds(pid * tile_t, tile_t), :],
                sem.at[slot, b],
            ).wait()


def kernel(inputs):
    n, t = inputs.shape
    tile_t = 256
    n_steps = t // tile_t
    out = pl.pallas_call(
        functools.partial(
            _pe_tile_kernel, tile_t=tile_t, n_steps=n_steps, n_batch=n
        ),
        grid=(n_steps,),
        out_specs=pl.BlockSpec(memory_space=pltpu.MemorySpace.HBM),
        out_shape=jax.ShapeDtypeStruct((n, t, _NUM_UNITS), jnp.float32),
        scratch_shapes=[
            pltpu.VMEM((2, tile_t, _NUM_UNITS), jnp.float32),
            pltpu.VMEM((tile_t, _NUM_UNITS), jnp.float32),
            pltpu.VMEM((tile_t, _NUM_UNITS), jnp.float32),
            pltpu.SemaphoreType.DMA((2, 4)),
        ],
    )()
    return out
